# baseline (speedup 1.0000x reference)
import functools

import jax
import jax.numpy as jnp
from jax.experimental import pallas as pl
from jax.experimental.pallas import tpu as pltpu

BT = 512


def _body(x_ref, p_ref, g_ref, o_ref, *, scale):
    acc = jax.lax.dot_general(
        x_ref[...], p_ref[...],
        dimension_numbers=(((1,), (1,)), ((), ())),
        preferred_element_type=jnp.float32,
    )
    o_ref[...] = jnp.maximum(acc * scale - g_ref[...], 0.0)


def kernel(x, proto_k, gate):
    t, d = x.shape
    n = proto_k.shape[0]
    scale = 1.0 / (d ** 0.5)
    gate2d = gate.reshape(1, n)
    grid = (t // BT,)
    out = pl.pallas_call(
        functools.partial(_body, scale=scale),
        grid=grid,
        in_specs=[
            pl.BlockSpec((BT, d), lambda i: (i, 0)),
            pl.BlockSpec((n, d), lambda i: (0, 0)),
            pl.BlockSpec((1, n), lambda i: (0, 0)),
        ],
        out_specs=pl.BlockSpec((BT, n), lambda i: (i, 0)),
        out_shape=jax.ShapeDtypeStruct((t, n), jnp.float32),
        compiler_params=pltpu.CompilerParams(
            dimension_semantics=("parallel",),
        ),
    )(x, proto_k, gate2d)
    return jnp.maximum(out, 0.0)


# speedup vs baseline: 1.0753x; 1.0753x over previous
import functools

import jax
import jax.numpy as jnp
from jax.experimental import pallas as pl
from jax.experimental.pallas import tpu as pltpu

BT = 512


def _body(x_ref, p_ref, g_ref, o_ref, *, scale):
    acc = jax.lax.dot_general(
        p_ref[...], x_ref[...],
        dimension_numbers=(((1,), (1,)), ((), ())),
        preferred_element_type=jnp.float32,
    )
    o_ref[...] = jnp.maximum(acc * scale - g_ref[...], 0.0)


def kernel(x, proto_k, gate):
    t, d = x.shape
    n = proto_k.shape[0]
    scale = 1.0 / (d ** 0.5)
    gate2d = gate.reshape(n, 1)
    grid = (t // BT,)
    out_t = pl.pallas_call(
        functools.partial(_body, scale=scale),
        grid=grid,
        in_specs=[
            pl.BlockSpec((BT, d), lambda i: (i, 0)),
            pl.BlockSpec((n, d), lambda i: (0, 0)),
            pl.BlockSpec((n, 1), lambda i: (0, 0)),
        ],
        out_specs=pl.BlockSpec((n, BT), lambda i: (0, i)),
        out_shape=jax.ShapeDtypeStruct((n, t), jnp.float32),
        compiler_params=pltpu.CompilerParams(
            dimension_semantics=("parallel",),
        ),
    )(x, proto_k, gate2d)
    return out_t.T


# speedup vs baseline: 1.1062x; 1.0287x over previous
import functools

import jax
import jax.numpy as jnp
from jax.experimental import pallas as pl
from jax.experimental.pallas import tpu as pltpu

BT = 512


def _body(x_ref, p_ref, g_ref, o_ref, *, scale):
    acc = jax.lax.dot_general(
        p_ref[...], x_ref[...],
        dimension_numbers=(((1,), (1,)), ((), ())),
        preferred_element_type=jnp.float32,
    )
    gate_col = g_ref[...].T
    o_ref[...] = jnp.maximum(acc * scale - gate_col, 0.0)


def kernel(x, proto_k, gate):
    t, d = x.shape
    n = proto_k.shape[0]
    scale = 1.0 / (d ** 0.5)
    gate2d = gate.reshape(1, n)
    grid = (t // BT,)
    out_t = pl.pallas_call(
        functools.partial(_body, scale=scale),
        grid=grid,
        in_specs=[
            pl.BlockSpec((BT, d), lambda i: (i, 0)),
            pl.BlockSpec((n, d), lambda i: (0, 0)),
            pl.BlockSpec((1, n), lambda i: (0, 0)),
        ],
        out_specs=pl.BlockSpec((n, BT), lambda i: (0, i)),
        out_shape=jax.ShapeDtypeStruct((n, t), jnp.float32),
        compiler_params=pltpu.CompilerParams(
            dimension_semantics=("parallel",),
        ),
    )(x, proto_k, gate2d)
    return out_t.T
